# Initial kernel scaffold; baseline (speedup 1.0000x reference)
import jax
import jax.numpy as jnp
from jax.experimental import pallas as pl


def kernel(low_frequency, high_frequency, params):
    raise NotImplementedError("write your pallas kernel here")


# speedup vs baseline: 7.0392x; 7.0392x over previous
import jax
import jax.numpy as jnp
from jax import lax
from jax.experimental import pallas as pl

N = 4096
D = 64
B = 128
S = 64
IN = B * S
H = 64
K = 32
ALPHA = 3.0
RB = 256


def _leaky(x):
    return jnp.where(x >= 0, x, 0.01 * x)


def _bn(x, g, b):
    m = jnp.mean(x, axis=0)
    v = jnp.var(x, axis=0)
    return g * (x - m) / jnp.sqrt(v + 1e-5) + b


def _dense_query(x, p):
    h = x @ p["W1"].T + p["b1"]
    h = _leaky(_bn(h, p["g1"], p["be1"]))
    h = h @ p["W2"].T + p["b2"]
    h = _leaky(_bn(h, p["g2"], p["be2"]))
    return h @ p["W3"].T + p["b3"]


def _node_embedding(low, high, p):
    lf = jnp.transpose(low, (2, 0, 1)).reshape(N, -1)
    hf = jnp.transpose(high, (2, 0, 1)).reshape(N, -1)
    ql = jax.nn.softmax(_dense_query(lf, p["ql"]), axis=-1)
    qh = jax.nn.softmax(_dense_query(hf, p["qh"]), axis=-1)
    node_low = _leaky(ql @ p["low_bank"])
    node_high = _leaky(qh @ p["high_bank"])
    return 3.0 * node_low + 3.0 * node_high


def _topk_mask_body(v1b_ref, v2b_ref, v1f_ref, v2f_ref, noise_ref, out_ref):
    mat = lax.dot_general(v1b_ref[...], v2f_ref[...],
                          (((1,), (1,)), ((), ())),
                          preferred_element_type=jnp.float32)
    mat -= lax.dot_general(v2b_ref[...], v1f_ref[...],
                           (((1,), (1,)), ((), ())),
                           preferred_element_type=jnp.float32)
    adj = jnp.maximum(jnp.tanh(ALPHA * mat), 0.0)
    s = adj + noise_ref[...]
    key = lax.bitcast_convert_type(s, jnp.int32)

    def srch(t, T):
        cand = T + (jnp.int32(1) << (jnp.int32(29) - t))
        cnt = jnp.sum((key >= cand).astype(jnp.float32), axis=-1,
                      keepdims=True)
        return jnp.where(cnt >= K, cand, T)

    T = lax.fori_loop(0, 30, srch, jnp.zeros((RB, 1), jnp.int32),
                      unroll=True)

    gt = key > T
    cgt = jnp.sum(gt.astype(jnp.float32), axis=-1, keepdims=True)
    need = jnp.float32(K) - cgt
    eq = key == T
    eqf = eq.astype(jnp.float32)
    col = lax.broadcasted_iota(jnp.int32, (RB, N), 1)

    def tsrch(t, Jc):
        cand = Jc + (jnp.int32(1) << (jnp.int32(11) - t))
        c = jnp.sum(jnp.where(col <= cand, eqf, 0.0), axis=-1,
                    keepdims=True)
        return jnp.where(c < need, cand, Jc)

    J = lax.fori_loop(0, 12, tsrch, jnp.full((RB, 1), -1, jnp.int32),
                      unroll=True)

    mask = gt | (eq & (col <= (J + 1)))
    out_ref[...] = jnp.where(mask, adj, 0.0)


def _topk_mask(v1, v2, noise):
    return pl.pallas_call(
        _topk_mask_body,
        grid=(N // RB,),
        in_specs=[
            pl.BlockSpec((RB, D), lambda i: (i, 0)),
            pl.BlockSpec((RB, D), lambda i: (i, 0)),
            pl.BlockSpec((N, D), lambda i: (0, 0)),
            pl.BlockSpec((N, D), lambda i: (0, 0)),
            pl.BlockSpec((RB, N), lambda i: (i, 0)),
        ],
        out_specs=pl.BlockSpec((RB, N), lambda i: (i, 0)),
        out_shape=jax.ShapeDtypeStruct((N, N), jnp.float32),
    )(v1, v2, v1, v2, noise)


def kernel(low_frequency, high_frequency, params):
    vec1 = _node_embedding(low_frequency, high_frequency, params["emb1"])
    vec2 = _node_embedding(low_frequency, high_frequency, params["emb2"])
    noise = jax.random.uniform(jax.random.key(42), (N, N), jnp.float32) * 0.01
    return _topk_mask(vec1, vec2, noise)
